# baseline (speedup 1.0000x reference)
import functools

import numpy as np

import jax
import jax.numpy as jnp
from jax import lax
from jax.experimental import pallas as pl
from jax.experimental.pallas import tpu as pltpu
from jax.experimental.pallas import tpu_sc as plsc

NC = 2
NS = 16
NW = NC * NS
L = 16

N = 4_000_000
NVEC = N // L
BASE_V = NVEC // NW
EXTRA4 = (NVEC - BASE_V * NW) // 4

CH_V = 512
CH_E = CH_V * L
NFULL = BASE_V // CH_V
TAIL_LO = BASE_V - NFULL * CH_V
TAIL_HI = TAIL_LO + 4

SENT = 4

TAB = 128

TC_ROWS = N // 128
TC_BLK = 4096
TC_GRID = (TC_ROWS + TC_BLK - 1) // TC_BLK


def _motif_masks():
    m = [np.zeros((TAB,), np.float32) for _ in range(3)]
    for code in range(100):
        x, r = divmod(code, 20)
        y, cur = divmod(r, 4)
        if cur != 1:
            continue
        wx = x in (0, 3)
        ry = y in (0, 2)
        sx = x in (1, 2)
        yy = y in (1, 3)
        if sx and yy:
            m[2][code] = 1.0
        elif wx and ry:
            m[1][code] = 1.0
        else:
            m[0][code] = 1.0
    return m

_M0, _M1, _M2 = _motif_masks()


def _ivec(c):
    return lax.iota(jnp.int32, L) * 0 + c


def _build_table(mp_ref, tab_ref):
    zf = _ivec(0).astype(jnp.float32)
    mp = mp_ref[...]
    p0 = zf + mp[0]
    p1 = zf + mp[1]
    p2 = zf + mp[2]
    for j in range(TAB // L):
        code = lax.iota(jnp.int32, L) + (j * L)
        x = code // 20
        r = code - x * 20
        y = r // 4
        cur = r - y * 4
        wx = (x == 0) | (x == 3)
        ry = (y == 0) | (y == 2)
        sx = (x == 1) | (x == 2)
        yy = (y == 1) | (y == 3)
        val = jnp.where(cur == 1,
                        jnp.where(sx & yy, p2,
                                  jnp.where(wx & ry, p1, p0)),
                        zf)
        tab_ref[pl.ds(j * L, L)] = val


def _worker_layout():
    cid = lax.axis_index("c")
    sid = lax.axis_index("s")
    wid = sid * NC + cid
    start0 = (BASE_V * wid + 4 * jnp.minimum(wid, EXTRA4)) * L
    return wid, start0


def _codes(buf, b):
    c2 = buf[pl.ds(b + 6, L)]
    c1 = buf[pl.ds(b + 7, L)]
    c0 = buf[pl.ds(b + 8, L)]
    return c2 * 20 + c1 * 4 + c0


def _patch_sentinel(buf, wid):
    @pl.when(wid == 0)
    def _():
        idx = lax.iota(jnp.int32, L)
        plsc.store_scatter(buf, (idx,), _ivec(SENT),
                           mask=(idx >= 6) & (idx < 8))


def _start_in(seq_hbm, buf, start, wid, k, sem):
    if k == 0:
        shift = (wid == 0).astype(jnp.int32) * 8
        return pltpu.async_copy(
            seq_hbm.at[pl.ds(start - 8 + shift, CH_E + 8)],
            buf.at[pl.ds(shift, CH_E + 8)], sem)
    return pltpu.async_copy(seq_hbm.at[pl.ds(start - 8, CH_E + 8)],
                            buf.at[pl.ds(0, CH_E + 8)], sem)


def _tail_in(seq_hbm, buf, tail_start, wid):
    @pl.when(wid < EXTRA4)
    def _():
        pltpu.sync_copy(seq_hbm.at[pl.ds(tail_start - 8, TAIL_HI * L + 8)],
                        buf.at[pl.ds(0, TAIL_HI * L + 8)])
    @pl.when(wid >= EXTRA4)
    def _():
        pltpu.sync_copy(seq_hbm.at[pl.ds(tail_start - 8, TAIL_LO * L + 8)],
                        buf.at[pl.ds(0, TAIL_LO * L + 8)])


def _main_pass(seq_hbm, mp_hbm, out_hbm, part_hbm,
               buf0, buf1, obuf0, obuf1, mpv, tabv, accv,
               isem0, isem1, osem0, osem1):
    wid, start0 = _worker_layout()
    bufs = (buf0, buf1)
    obufs = (obuf0, obuf1)
    isems = (isem0, isem1)
    osems = (osem0, osem1)
    pltpu.sync_copy(mp_hbm, mpv)

    _patch_sentinel(buf0, wid)
    h_in = {0: _start_in(seq_hbm, buf0, start0, wid, 0, isem0)}
    _build_table(mpv, tabv)

    def make_body(buf, obuf):
        def body(i, acc):
            b = i * (4 * L)
            vs = []
            for u in range(4):
                v = plsc.load_gather(tabv, (_codes(buf, b + u * L),))
                obuf[pl.ds(b + u * L, L)] = v
                vs.append(v)
            return acc + (vs[0] + vs[1]) + (vs[2] + vs[3])
        return body

    acc = _ivec(0).astype(jnp.float32)
    h_out = {}
    for k in range(NFULL):
        start = start0 + k * CH_E
        if k + 1 < NFULL:
            h_in[k + 1] = _start_in(seq_hbm, bufs[(k + 1) % 2],
                                    start0 + (k + 1) * CH_E, wid, k + 1,
                                    isems[(k + 1) % 2])
        h_in.pop(k).wait()
        if k - 2 in h_out:
            h_out.pop(k - 2).wait()
        acc = lax.fori_loop(0, CH_V // 4,
                            make_body(bufs[k % 2], obufs[k % 2]), acc)
        h_out[k] = pltpu.async_copy(obufs[k % 2],
                                    out_hbm.at[pl.ds(start, CH_E)],
                                    osems[k % 2])

    tail_start = start0 + NFULL * CH_E
    _tail_in(seq_hbm, buf1, tail_start, wid)
    if NFULL - 2 in h_out:
        h_out.pop(NFULL - 2).wait()
    tail_nv4 = TAIL_LO // 4 + (wid < EXTRA4).astype(jnp.int32)
    acc = lax.fori_loop(0, tail_nv4, make_body(buf1, obuf1), acc)
    @pl.when(wid < EXTRA4)
    def _():
        pltpu.sync_copy(obuf1.at[pl.ds(0, TAIL_HI * L)],
                        out_hbm.at[pl.ds(tail_start, TAIL_HI * L)])
    @pl.when(wid >= EXTRA4)
    def _():
        pltpu.sync_copy(obuf1.at[pl.ds(0, TAIL_LO * L)],
                        out_hbm.at[pl.ds(tail_start, TAIL_LO * L)])

    accv[...] = acc
    pltpu.sync_copy(accv, part_hbm.at[pl.ds(wid * L, L)])
    for k in sorted(h_out):
        h_out.pop(k).wait()


def _scale_body(part_ref, u_ref, o_ref):
    inv = 1.0 / jnp.sum(part_ref[...])
    o_ref[...] = u_ref[...] * inv


@functools.cache
def _calls():
    mesh = plsc.VectorSubcoreMesh(core_axis_name="c", subcore_axis_name="s",
                                  num_cores=NC, num_subcores=NS)
    main_pass = pl.kernel(
        _main_pass,
        out_type=(jax.ShapeDtypeStruct((N,), jnp.float32),
                  jax.ShapeDtypeStruct((NW * L,), jnp.float32)),
        mesh=mesh,
        scratch_types=[
            pltpu.VMEM((16 + CH_E,), jnp.int32),
            pltpu.VMEM((16 + CH_E,), jnp.int32),
            pltpu.VMEM((CH_E,), jnp.float32),
            pltpu.VMEM((CH_E,), jnp.float32),
            pltpu.VMEM((L,), jnp.float32),
            pltpu.VMEM((TAB,), jnp.float32),
            pltpu.VMEM((L,), jnp.float32),
            pltpu.SemaphoreType.DMA,
            pltpu.SemaphoreType.DMA,
            pltpu.SemaphoreType.DMA,
            pltpu.SemaphoreType.DMA,
        ],
        compiler_params=pltpu.CompilerParams(needs_layout_passes=False),
    )
    scale_pass = pl.pallas_call(
        _scale_body,
        grid=(TC_GRID,),
        in_specs=[
            pl.BlockSpec((NW * L // 128, 128), lambda j: (0, 0)),
            pl.BlockSpec((TC_BLK, 128), lambda j: (j, 0)),
        ],
        out_specs=pl.BlockSpec((TC_BLK, 128), lambda j: (j, 0)),
        out_shape=jax.ShapeDtypeStruct((TC_ROWS, 128), jnp.float32),
    )
    return main_pass, scale_pass


def kernel(sequence, motifs_prob):
    main_pass, scale_pass = _calls()
    mp_pad = jnp.zeros((L,), jnp.float32).at[:3].set(motifs_prob)
    unnorm, parts = main_pass(sequence, mp_pad)
    out = scale_pass(parts.reshape(NW * L // 128, 128),
                     unnorm.reshape(TC_ROWS, 128))
    return out.reshape(N)


# speedup vs baseline: 1.2799x; 1.2799x over previous
import functools

import numpy as np

import jax
import jax.numpy as jnp
from jax import lax
from jax.experimental import pallas as pl
from jax.experimental.pallas import tpu as pltpu
from jax.experimental.pallas import tpu_sc as plsc

NC = 2
NS = 16
NW = NC * NS
L = 16

N = 4_000_000
NVEC = N // L
BASE_V = NVEC // NW
EXTRA4 = (NVEC - BASE_V * NW) // 4

CH_V = 512
CH_E = CH_V * L
NFULL = BASE_V // CH_V
TAIL_LO = BASE_V - NFULL * CH_V
TAIL_HI = TAIL_LO + 4

SENT = 4

TAB = 128

TC_ROWS = N // 128
TC_BLK = 4096
TC_GRID = (TC_ROWS + TC_BLK - 1) // TC_BLK


def _motif_masks():
    m = [np.zeros((TAB,), np.float32) for _ in range(3)]
    for code in range(100):
        x, r = divmod(code, 20)
        y, cur = divmod(r, 4)
        if cur != 1:
            continue
        wx = x in (0, 3)
        ry = y in (0, 2)
        sx = x in (1, 2)
        yy = y in (1, 3)
        if sx and yy:
            m[2][code] = 1.0
        elif wx and ry:
            m[1][code] = 1.0
        else:
            m[0][code] = 1.0
    return m

_M0, _M1, _M2 = _motif_masks()


def _ivec(c):
    return lax.iota(jnp.int32, L) * 0 + c


def _build_table(mp_ref, tab_ref):
    zf = _ivec(0).astype(jnp.float32)
    mp = mp_ref[...]
    p0 = zf + mp[0]
    p1 = zf + mp[1]
    p2 = zf + mp[2]
    for j in range(TAB // L):
        code = lax.iota(jnp.int32, L) + (j * L)
        x = code // 20
        r = code - x * 20
        y = r // 4
        cur = r - y * 4
        wx = (x == 0) | (x == 3)
        ry = (y == 0) | (y == 2)
        sx = (x == 1) | (x == 2)
        yy = (y == 1) | (y == 3)
        val = jnp.where(cur == 1,
                        jnp.where(sx & yy, p2,
                                  jnp.where(wx & ry, p1, p0)),
                        zf)
        tab_ref[pl.ds(j * L, L)] = val


def _worker_layout():
    cid = lax.axis_index("c")
    sid = lax.axis_index("s")
    wid = sid * NC + cid
    start0 = (BASE_V * wid + 4 * jnp.minimum(wid, EXTRA4)) * L
    return wid, start0


def _codes(buf, b):
    c2 = buf[pl.ds(b + 6, L)]
    c1 = buf[pl.ds(b + 7, L)]
    c0 = buf[pl.ds(b + 8, L)]
    return c2 * 20 + c1 * 4 + c0


def _patch_sentinel(buf, wid):
    @pl.when(wid == 0)
    def _():
        idx = lax.iota(jnp.int32, L)
        plsc.store_scatter(buf, (idx,), _ivec(SENT),
                           mask=(idx >= 6) & (idx < 8))


def _start_in(seq_hbm, buf, start, wid, k, sem):
    if k == 0:
        shift = (wid == 0).astype(jnp.int32) * 8
        return pltpu.async_copy(
            seq_hbm.at[pl.ds(start - 8 + shift, CH_E + 8)],
            buf.at[pl.ds(shift, CH_E + 8)], sem)
    return pltpu.async_copy(seq_hbm.at[pl.ds(start - 8, CH_E + 8)],
                            buf.at[pl.ds(0, CH_E + 8)], sem)


def _tail_in(seq_hbm, buf, tail_start, wid):
    @pl.when(wid < EXTRA4)
    def _():
        pltpu.sync_copy(seq_hbm.at[pl.ds(tail_start - 8, TAIL_HI * L + 8)],
                        buf.at[pl.ds(0, TAIL_HI * L + 8)])
    @pl.when(wid >= EXTRA4)
    def _():
        pltpu.sync_copy(seq_hbm.at[pl.ds(tail_start - 8, TAIL_LO * L + 8)],
                        buf.at[pl.ds(0, TAIL_LO * L + 8)])


def _main_pass(seq_hbm, mp_hbm, out_hbm, part_hbm,
               buf0, buf1, obuf0, obuf1, mpv, tabv, accv,
               isem0, isem1, osem0, osem1):
    wid, start0 = _worker_layout()
    bufs = (buf0, buf1)
    obufs = (obuf0, obuf1)
    isems = (isem0, isem1)
    osems = (osem0, osem1)
    pltpu.sync_copy(mp_hbm, mpv)

    _patch_sentinel(buf0, wid)
    h_in = {0: _start_in(seq_hbm, buf0, start0, wid, 0, isem0)}
    _build_table(mpv, tabv)

    def make_body(buf, obuf):
        def body(i, acc):
            b = i * (2 * L)
            v0 = plsc.load_gather(tabv, (_codes(buf, b),))
            v1 = plsc.load_gather(tabv, (_codes(buf, b + L),))
            obuf[pl.ds(b, L)] = v0
            obuf[pl.ds(b + L, L)] = v1
            return acc + v0 + v1
        return body

    acc = _ivec(0).astype(jnp.float32)
    h_out = {}
    for k in range(NFULL):
        start = start0 + k * CH_E
        if k + 1 < NFULL:
            h_in[k + 1] = _start_in(seq_hbm, bufs[(k + 1) % 2],
                                    start0 + (k + 1) * CH_E, wid, k + 1,
                                    isems[(k + 1) % 2])
        h_in.pop(k).wait()
        if k - 2 in h_out:
            h_out.pop(k - 2).wait()
        acc = lax.fori_loop(0, CH_V // 2,
                            make_body(bufs[k % 2], obufs[k % 2]), acc)
        h_out[k] = pltpu.async_copy(obufs[k % 2],
                                    out_hbm.at[pl.ds(start, CH_E)],
                                    osems[k % 2])

    tail_start = start0 + NFULL * CH_E
    _tail_in(seq_hbm, buf1, tail_start, wid)
    if NFULL - 2 in h_out:
        h_out.pop(NFULL - 2).wait()
    tail_nv2 = TAIL_LO // 2 + 2 * (wid < EXTRA4).astype(jnp.int32)
    acc = lax.fori_loop(0, tail_nv2, make_body(buf1, obuf1), acc)
    @pl.when(wid < EXTRA4)
    def _():
        pltpu.sync_copy(obuf1.at[pl.ds(0, TAIL_HI * L)],
                        out_hbm.at[pl.ds(tail_start, TAIL_HI * L)])
    @pl.when(wid >= EXTRA4)
    def _():
        pltpu.sync_copy(obuf1.at[pl.ds(0, TAIL_LO * L)],
                        out_hbm.at[pl.ds(tail_start, TAIL_LO * L)])

    accv[...] = acc
    pltpu.sync_copy(accv, part_hbm.at[pl.ds(wid * L, L)])
    for k in sorted(h_out):
        h_out.pop(k).wait()


def _scale_body(part_ref, u_ref, o_ref):
    inv = 1.0 / jnp.sum(part_ref[...])
    o_ref[...] = u_ref[...] * inv


@functools.cache
def _calls():
    mesh = plsc.VectorSubcoreMesh(core_axis_name="c", subcore_axis_name="s",
                                  num_cores=NC, num_subcores=NS)
    main_pass = pl.kernel(
        _main_pass,
        out_type=(jax.ShapeDtypeStruct((N,), jnp.float32),
                  jax.ShapeDtypeStruct((NW * L,), jnp.float32)),
        mesh=mesh,
        scratch_types=[
            pltpu.VMEM((16 + CH_E,), jnp.int32),
            pltpu.VMEM((16 + CH_E,), jnp.int32),
            pltpu.VMEM((CH_E,), jnp.float32),
            pltpu.VMEM((CH_E,), jnp.float32),
            pltpu.VMEM((L,), jnp.float32),
            pltpu.VMEM((TAB,), jnp.float32),
            pltpu.VMEM((L,), jnp.float32),
            pltpu.SemaphoreType.DMA,
            pltpu.SemaphoreType.DMA,
            pltpu.SemaphoreType.DMA,
            pltpu.SemaphoreType.DMA,
        ],
        compiler_params=pltpu.CompilerParams(needs_layout_passes=False),
    )
    scale_pass = pl.pallas_call(
        _scale_body,
        grid=(TC_GRID,),
        in_specs=[
            pl.BlockSpec((NW * L // 128, 128), lambda j: (0, 0)),
            pl.BlockSpec((TC_BLK, 128), lambda j: (j, 0)),
        ],
        out_specs=pl.BlockSpec((TC_BLK, 128), lambda j: (j, 0)),
        out_shape=jax.ShapeDtypeStruct((TC_ROWS, 128), jnp.float32),
    )
    return main_pass, scale_pass


def kernel(sequence, motifs_prob):
    main_pass, scale_pass = _calls()
    mp_pad = jnp.zeros((L,), jnp.float32).at[:3].set(motifs_prob)
    unnorm, parts = main_pass(sequence, mp_pad)
    out = scale_pass(parts.reshape(NW * L // 128, 128),
                     unnorm.reshape(TC_ROWS, 128))
    return out.reshape(N)


# speedup vs baseline: 1.2980x; 1.0141x over previous
import functools

import numpy as np

import jax
import jax.numpy as jnp
from jax import lax
from jax.experimental import pallas as pl
from jax.experimental.pallas import tpu as pltpu
from jax.experimental.pallas import tpu_sc as plsc

NC = 2
NS = 16
NW = NC * NS
L = 16

N = 4_000_000
NVEC = N // L
BASE_V = NVEC // NW
EXTRA4 = (NVEC - BASE_V * NW) // 4

CH_V = 512
CH_E = CH_V * L
NFULL = BASE_V // CH_V
TAIL_LO = BASE_V - NFULL * CH_V
TAIL_HI = TAIL_LO + 4

SENT = 4

TAB = 128

TC_ROWS = N // 128
TC_BLK = 4096
TC_GRID = (TC_ROWS + TC_BLK - 1) // TC_BLK


def _motif_masks():
    m = [np.zeros((TAB,), np.float32) for _ in range(3)]
    for code in range(100):
        x, r = divmod(code, 20)
        y, cur = divmod(r, 4)
        if cur != 1:
            continue
        wx = x in (0, 3)
        ry = y in (0, 2)
        sx = x in (1, 2)
        yy = y in (1, 3)
        if sx and yy:
            m[2][code] = 1.0
        elif wx and ry:
            m[1][code] = 1.0
        else:
            m[0][code] = 1.0
    return m

_M0, _M1, _M2 = _motif_masks()


def _ivec(c):
    return lax.iota(jnp.int32, L) * 0 + c


def _build_table(mp_ref, tab_ref):
    zf = _ivec(0).astype(jnp.float32)
    mp = mp_ref[...]
    p0 = zf + mp[0]
    p1 = zf + mp[1]
    p2 = zf + mp[2]
    for j in range(TAB // L):
        code = lax.iota(jnp.int32, L) + (j * L)
        x = code // 20
        r = code - x * 20
        y = r // 4
        cur = r - y * 4
        wx = (x == 0) | (x == 3)
        ry = (y == 0) | (y == 2)
        sx = (x == 1) | (x == 2)
        yy = (y == 1) | (y == 3)
        val = jnp.where(cur == 1,
                        jnp.where(sx & yy, p2,
                                  jnp.where(wx & ry, p1, p0)),
                        zf)
        tab_ref[pl.ds(j * L, L)] = val


def _worker_layout():
    cid = lax.axis_index("c")
    sid = lax.axis_index("s")
    wid = sid * NC + cid
    start0 = (BASE_V * wid + 4 * jnp.minimum(wid, EXTRA4)) * L
    return wid, start0


def _codes(buf, b):
    c2 = buf[pl.ds(b + 6, L)]
    c1 = buf[pl.ds(b + 7, L)]
    c0 = buf[pl.ds(b + 8, L)]
    return c2 * 20 + c1 * 4 + c0


def _patch_sentinel(buf, wid):
    @pl.when(wid == 0)
    def _():
        idx = lax.iota(jnp.int32, L)
        plsc.store_scatter(buf, (idx,), _ivec(SENT),
                           mask=(idx >= 6) & (idx < 8))


def _start_in(seq_hbm, buf, start, wid, k, sem):
    if k == 0:
        shift = (wid == 0).astype(jnp.int32) * 8
        return pltpu.async_copy(
            seq_hbm.at[pl.ds(start - 8 + shift, CH_E + 8)],
            buf.at[pl.ds(shift, CH_E + 8)], sem)
    return pltpu.async_copy(seq_hbm.at[pl.ds(start - 8, CH_E + 8)],
                            buf.at[pl.ds(0, CH_E + 8)], sem)


def _tail_in(seq_hbm, buf, tail_start, wid):
    @pl.when(wid < EXTRA4)
    def _():
        pltpu.sync_copy(seq_hbm.at[pl.ds(tail_start - 8, TAIL_HI * L + 8)],
                        buf.at[pl.ds(0, TAIL_HI * L + 8)])
    @pl.when(wid >= EXTRA4)
    def _():
        pltpu.sync_copy(seq_hbm.at[pl.ds(tail_start - 8, TAIL_LO * L + 8)],
                        buf.at[pl.ds(0, TAIL_LO * L + 8)])


def _main_pass(seq_hbm, mp_hbm, out_hbm, part_hbm,
               buf0, buf1, obuf0, obuf1, mpv, tabv, accv,
               isem0, isem1, osem0, osem1):
    wid, start0 = _worker_layout()
    bufs = (buf0, buf1)
    obufs = (obuf0, obuf1)
    isems = (isem0, isem1)
    osems = (osem0, osem1)
    pltpu.sync_copy(mp_hbm, mpv.at[pl.ds(0, 3)])

    _patch_sentinel(buf0, wid)
    h_in = {0: _start_in(seq_hbm, buf0, start0, wid, 0, isem0)}
    _build_table(mpv, tabv)

    def make_body(buf, obuf):
        def body(i, acc):
            b = i * (2 * L)
            v0 = plsc.load_gather(tabv, (_codes(buf, b),))
            v1 = plsc.load_gather(tabv, (_codes(buf, b + L),))
            obuf[pl.ds(b, L)] = v0
            obuf[pl.ds(b + L, L)] = v1
            return acc + v0 + v1
        return body

    acc = _ivec(0).astype(jnp.float32)
    h_out = {}
    for k in range(NFULL):
        start = start0 + k * CH_E
        if k + 1 < NFULL:
            h_in[k + 1] = _start_in(seq_hbm, bufs[(k + 1) % 2],
                                    start0 + (k + 1) * CH_E, wid, k + 1,
                                    isems[(k + 1) % 2])
        h_in.pop(k).wait()
        if k - 2 in h_out:
            h_out.pop(k - 2).wait()
        acc = lax.fori_loop(0, CH_V // 2,
                            make_body(bufs[k % 2], obufs[k % 2]), acc)
        h_out[k] = pltpu.async_copy(obufs[k % 2],
                                    out_hbm.at[pl.ds(start, CH_E)],
                                    osems[k % 2])

    tail_start = start0 + NFULL * CH_E
    _tail_in(seq_hbm, buf1, tail_start, wid)
    if NFULL - 2 in h_out:
        h_out.pop(NFULL - 2).wait()
    tail_nv2 = TAIL_LO // 2 + 2 * (wid < EXTRA4).astype(jnp.int32)
    acc = lax.fori_loop(0, tail_nv2, make_body(buf1, obuf1), acc)
    @pl.when(wid < EXTRA4)
    def _():
        pltpu.sync_copy(obuf1.at[pl.ds(0, TAIL_HI * L)],
                        out_hbm.at[pl.ds(tail_start, TAIL_HI * L)])
    @pl.when(wid >= EXTRA4)
    def _():
        pltpu.sync_copy(obuf1.at[pl.ds(0, TAIL_LO * L)],
                        out_hbm.at[pl.ds(tail_start, TAIL_LO * L)])

    accv[...] = acc
    pltpu.sync_copy(accv, part_hbm.at[pl.ds(wid * L, L)])
    for k in sorted(h_out):
        h_out.pop(k).wait()


def _scale_body(part_ref, u_ref, o_ref):
    inv = 1.0 / jnp.sum(part_ref[...])
    o_ref[...] = u_ref[...] * inv


@functools.cache
def _calls():
    mesh = plsc.VectorSubcoreMesh(core_axis_name="c", subcore_axis_name="s",
                                  num_cores=NC, num_subcores=NS)
    main_pass = pl.kernel(
        _main_pass,
        out_type=(jax.ShapeDtypeStruct((N,), jnp.float32),
                  jax.ShapeDtypeStruct((NW * L,), jnp.float32)),
        mesh=mesh,
        scratch_types=[
            pltpu.VMEM((16 + CH_E,), jnp.int32),
            pltpu.VMEM((16 + CH_E,), jnp.int32),
            pltpu.VMEM((CH_E,), jnp.float32),
            pltpu.VMEM((CH_E,), jnp.float32),
            pltpu.VMEM((L,), jnp.float32),
            pltpu.VMEM((TAB,), jnp.float32),
            pltpu.VMEM((L,), jnp.float32),
            pltpu.SemaphoreType.DMA,
            pltpu.SemaphoreType.DMA,
            pltpu.SemaphoreType.DMA,
            pltpu.SemaphoreType.DMA,
        ],
        compiler_params=pltpu.CompilerParams(needs_layout_passes=False),
    )
    scale_pass = pl.pallas_call(
        _scale_body,
        grid=(TC_GRID,),
        in_specs=[
            pl.BlockSpec((NW * L // 128, 128), lambda j: (0, 0)),
            pl.BlockSpec((TC_BLK * 128,), lambda j: (j,)),
        ],
        out_specs=pl.BlockSpec((TC_BLK * 128,), lambda j: (j,)),
        out_shape=jax.ShapeDtypeStruct((N,), jnp.float32),
    )
    return main_pass, scale_pass


def kernel(sequence, motifs_prob):
    main_pass, scale_pass = _calls()
    unnorm, parts = main_pass(sequence, motifs_prob)
    return scale_pass(parts.reshape(NW * L // 128, 128), unnorm)
